# baseline (speedup 1.0000x reference)
import functools

import jax
import jax.numpy as jnp
from jax import lax
from jax.experimental import pallas as pl
from jax.experimental.pallas import tpu as pltpu
from jax.experimental.pallas import tpu_sc as plsc

D = 128
G = 64
HALF = 64
W = 64
TILES = 16
NB = 512


@functools.lru_cache(maxsize=None)
def _make_edge_kernel(NP, EP):
    per_tile = EP // TILES
    chunks = per_tile // W
    assert chunks % 2 == 0 and chunks >= 4
    rows_pt = NP // TILES
    mesh = plsc.VectorSubcoreMesh(core_axis_name="c", subcore_axis_name="s")
    out_t = (jax.ShapeDtypeStruct((NP, 2, HALF), jnp.float32),
             jax.ShapeDtypeStruct((NP, 2, HALF), jnp.float32))
    idx_t = pltpu.VMEM((W,), jnp.int32)
    ctr_t = pltpu.VMEM((W, HALF), jnp.float32)
    scratch = (
        [idx_t] * 4 +
        [idx_t] * 8 +
        [pltpu.VMEM((W, 2 * HALF), jnp.float32)] * 2 +
        [ctr_t] * 2 +
        [ctr_t] * 4 +
        [pltpu.VMEM((64, HALF), jnp.float32)] +
        [pltpu.VMEM_SHARED((NP, HALF), jnp.float32)] * 2 +
        [pltpu.SemaphoreType.DMA] * 6
    )

    @functools.partial(
        pl.kernel, out_type=out_t, mesh=mesh, scratch_types=scratch,
        compiler_params=pltpu.CompilerParams(use_tc_tiling_on_sc=False))
    def edge_kernel(srcp, dstp, ebt2, dx2, num_out, den_out,
                    isr0, isr1, idr0, idr1,
                    i2s0, i2s1, i2d0, i2d1, sd0, sd1, ssd0, ssd1,
                    rEB0, rEB1, rD0, rD1,
                    cn0, cn1, cd0, cd1, zb,
                    accn, accd, semG0, semG1, semI0, semI1, semS0, semS1):
        israw = (isr0, isr1)
        idraw = (idr0, idr1)
        i2s = (i2s0, i2s1)
        i2d = (i2d0, i2d1)
        sd = (sd0, sd1)
        ssd = (ssd0, ssd1)
        rEB = (rEB0, rEB1)
        rD = (rD0, rD1)
        cn = (cn0, cn1)
        cd = (cd0, cd1)
        semG = (semG0, semG1)
        semI = (semI0, semI1)
        semS = (semS0, semS1)

        c = lax.axis_index("c")
        s = lax.axis_index("s")

        zv = jnp.zeros((16,), jnp.float32)

        def zrow(i, _):
            for f in range(HALF // 16):
                zb[i, pl.ds(f * 16, 16)] = zv
            return 0

        lax.fori_loop(0, 64, zrow, 0)
        r0 = s * rows_pt
        for k in range(rows_pt // 64):
            pltpu.sync_copy(zb, accn.at[pl.ds(r0 + k * 64, 64)])
            pltpu.sync_copy(zb, accd.at[pl.ds(r0 + k * 64, 64)])
        plsc.subcore_barrier()

        base0 = s * per_tile

        def prep(b):
            for q in range(W // 16):
                sl = pl.ds(q * 16, 16)
                dv = idraw[b][sl]
                i2s[b][sl] = israw[b][sl] * 2 + c
                i2d[b][sl] = dv * 2 + c
                sd[b][sl] = dv

        def issue_gathers(b):
            pltpu.async_copy(ebt2.at[i2s[b]], rEB[b], semG[b])
            pltpu.async_copy(dx2.at[i2d[b]], rD[b], semG[b])

        def issue_idx(b, g):
            off = pl.multiple_of(base0 + g * W, W)
            pltpu.async_copy(srcp.at[pl.ds(off, W)], israw[b], semI[b])
            pltpu.async_copy(dstp.at[pl.ds(off, W)], idraw[b], semI[b])

        def drain_gathers(b):
            pltpu.make_async_copy(ebt2.at[pl.ds(0, W)], rEB[b], semG[b]).wait()
            pltpu.make_async_copy(dx2.at[pl.ds(0, W)], rD[b], semG[b]).wait()

        def drain_idx(b):
            pltpu.make_async_copy(srcp.at[pl.ds(0, W)], israw[b], semI[b]).wait()
            pltpu.make_async_copy(dstp.at[pl.ds(0, W)], idraw[b], semI[b]).wait()

        def drain_scatters(b):
            pltpu.make_async_copy(num_out.at[pl.ds(0, W), 0], cn[b],
                                  semS[b]).wait()
            pltpu.make_async_copy(num_out.at[pl.ds(0, W), 0], cd[b],
                                  semS[b]).wait()

        def compute(b):
            def ew(w, _):
                for f in range(HALF // 16):
                    sl = pl.ds(f * 16, 16)
                    t = rD[b][w, sl] + rEB[b][w, sl]
                    sg = 1.0 / (1.0 + jnp.exp(-t))
                    cd[b][w, sl] = sg
                    cn[b][w, sl] = sg * rEB[b][w, pl.ds(HALF + f * 16, 16)]
                return 0

            lax.fori_loop(0, W, ew, 0)

        for b in (0, 1):
            off = pl.multiple_of(base0 + b * W, W)
            pltpu.sync_copy(srcp.at[pl.ds(off, W)], israw[b])
            pltpu.sync_copy(dstp.at[pl.ds(off, W)], idraw[b])
            prep(b)
            issue_gathers(b)
        issue_idx(0, 2)
        issue_idx(1, 3)

        def pair(p, _):
            for b in (0, 1):
                g = p * 2 + b
                drain_gathers(b)

                @pl.when(g >= 2)
                def _():
                    drain_scatters(b)

                compute(b)
                for q in range(W // 16):
                    sl = pl.ds(q * 16, 16)
                    ssd[b][sl] = sd[b][sl]
                pltpu.async_copy(cn[b], accn.at[ssd[b]], semS[b], add=True)
                pltpu.async_copy(cd[b], accd.at[ssd[b]], semS[b], add=True)

                @pl.when(g + 2 < chunks)
                def _():
                    drain_idx(b)
                    prep(b)
                    issue_gathers(b)

                @pl.when(g + 4 < chunks)
                def _():
                    issue_idx(b, g + 4)
            return 0

        lax.fori_loop(0, chunks // 2, pair, 0)
        drain_scatters(0)
        drain_scatters(1)
        plsc.subcore_barrier()

        for k in range(rows_pt // 64):
            rr = pl.ds(r0 + k * 64, 64)
            pltpu.sync_copy(accn.at[rr], num_out.at[rr, c])
            pltpu.sync_copy(accd.at[rr], den_out.at[rr, c])

    return edge_kernel


def _split_tables(y):
    return y[:, 0 * D:1 * D], y[:, 1 * D:2 * D], y[:, 2 * D:4 * D]


def _matmul4(h, wc, bc):
    NP = h.shape[0]

    def body(h_ref, w_ref, b_ref, oa, od, oeb):
        y = jnp.dot(h_ref[...], w_ref[...],
                    preferred_element_type=jnp.float32) + b_ref[...]
        ax, dx, ebt = _split_tables(y)
        oa[...] = ax
        od[...] = dx
        oeb[...] = ebt

    out = [jax.ShapeDtypeStruct((NP, D), jnp.float32)] * 2 + \
          [jax.ShapeDtypeStruct((NP, 2 * D), jnp.float32)]
    return pl.pallas_call(
        body, grid=(NP // NB,),
        in_specs=[pl.BlockSpec((NB, D), lambda i: (i, 0)),
                  pl.BlockSpec((D, 4 * D), lambda i: (0, 0)),
                  pl.BlockSpec((1, 4 * D), lambda i: (0, 0))],
        out_specs=[pl.BlockSpec((NB, D), lambda i: (i, 0))] * 2 +
                  [pl.BlockSpec((NB, 2 * D), lambda i: (i, 0))],
        out_shape=out)(h, wc, bc)


def _hn_and_mask(a_ref, n_ref, d_ref, i, n_valid):
    hn = a_ref[...] + n_ref[...] / (d_ref[...] + 1e-6)
    row = lax.broadcasted_iota(jnp.int32, (NB, 1), 0) + i * NB
    return hn, row < n_valid


def _bn_apply(x_ref, hn, mask, sums, sumsq, g_ref, b_ref, inv_n):
    mean = sums[...] * inv_n
    var = sumsq[...] * inv_n - mean * mean
    y = (hn - mean) * lax.rsqrt(var + 1e-5) * g_ref[...] + b_ref[...]
    h = x_ref[...] + jnp.maximum(y, 0.0)
    return jnp.where(mask, h, 0.0)


def _fused_mid(x, ax, num, den, bn_g, bn_b, wc, bc, n_valid):
    NP = x.shape[0]
    inv_n = 1.0 / n_valid

    def body(x_ref, a_ref, n_ref, d_ref, g_ref, b_ref, w_ref, bc_ref,
             oh, oa, od, oeb, sums, sumsq):
        ph = pl.program_id(0)
        i = pl.program_id(1)
        hn, mask = _hn_and_mask(a_ref, n_ref, d_ref, i, n_valid)

        @pl.when(ph == 0)
        def _():
            @pl.when(i == 0)
            def _():
                sums[...] = jnp.zeros_like(sums)
                sumsq[...] = jnp.zeros_like(sumsq)

            hm = jnp.where(mask, hn, 0.0)
            sums[...] += jnp.sum(hm, axis=0, keepdims=True)
            sumsq[...] += jnp.sum(hm * hm, axis=0, keepdims=True)

        @pl.when(ph == 1)
        def _():
            h = _bn_apply(x_ref, hn, mask, sums, sumsq, g_ref, b_ref, inv_n)
            oh[...] = h
            y = jnp.dot(h, w_ref[...],
                        preferred_element_type=jnp.float32) + bc_ref[...]
            ax2, dx2, ebt2 = _split_tables(y)
            oa[...] = ax2
            od[...] = dx2
            oeb[...] = ebt2

    blk = pl.BlockSpec((NB, D), lambda ph, i: (i, 0))
    one = pl.BlockSpec((1, D), lambda ph, i: (0, 0))
    out = [jax.ShapeDtypeStruct((NP, D), jnp.float32)] * 3 + \
          [jax.ShapeDtypeStruct((NP, 2 * D), jnp.float32)]
    return pl.pallas_call(
        body, grid=(2, NP // NB),
        in_specs=[blk] * 4 + [one] * 2 +
                 [pl.BlockSpec((D, 4 * D), lambda ph, i: (0, 0)),
                  pl.BlockSpec((1, 4 * D), lambda ph, i: (0, 0))],
        out_specs=[blk] * 3 +
                  [pl.BlockSpec((NB, 2 * D), lambda ph, i: (i, 0))],
        out_shape=out,
        scratch_shapes=[pltpu.VMEM((1, D), jnp.float32)] * 2,
    )(x, ax, num, den, bn_g, bn_b, wc, bc)


def _fused_final(x, ax, num, den, bn_g, bn_b, bids, w1, b1, w2, b2, n_valid):
    NP = x.shape[0]
    inv_n = 1.0 / n_valid
    nblk = NP // NB

    def body(x_ref, a_ref, n_ref, d_ref, g_ref, b_ref, bid_ref,
             w1_ref, b1_ref, w2_ref, b2_ref, out_ref, psum, pcnt,
             sums, sumsq):
        ph = pl.program_id(0)
        i = pl.program_id(1)
        hn, mask = _hn_and_mask(a_ref, n_ref, d_ref, i, n_valid)

        @pl.when(ph == 0)
        def _():
            @pl.when(i == 0)
            def _():
                sums[...] = jnp.zeros_like(sums)
                sumsq[...] = jnp.zeros_like(sumsq)

            hm = jnp.where(mask, hn, 0.0)
            sums[...] += jnp.sum(hm, axis=0, keepdims=True)
            sumsq[...] += jnp.sum(hm * hm, axis=0, keepdims=True)

        @pl.when(ph == 1)
        def _():
            h = _bn_apply(x_ref, hn, mask, sums, sumsq, g_ref, b_ref, inv_n)
            oh = (bid_ref[...] ==
                  lax.broadcasted_iota(jnp.int32, (1, G), 1))
            oh = oh.astype(jnp.float32)

            @pl.when(i == 0)
            def _():
                psum[...] = jnp.zeros_like(psum)
                pcnt[...] = jnp.zeros_like(pcnt)

            dn = (((0,), (0,)), ((), ()))
            psum[...] += lax.dot_general(oh, h, dn,
                                         preferred_element_type=jnp.float32)
            pcnt[...] += lax.dot_general(oh, jnp.ones((NB, D), jnp.float32),
                                         dn, preferred_element_type=jnp.float32)

            @pl.when(i == nblk - 1)
            def _():
                gm = psum[...] / jnp.maximum(pcnt[...], 1.0)
                t = jnp.dot(gm, w1_ref[...],
                            preferred_element_type=jnp.float32)
                t = jnp.maximum(t + b1_ref[...], 0.0)
                out_ref[...] = jnp.dot(
                    t, w2_ref[...],
                    preferred_element_type=jnp.float32) + b2_ref[...]

    blk = pl.BlockSpec((NB, D), lambda ph, i: (i, 0))
    one = pl.BlockSpec((1, D), lambda ph, i: (0, 0))
    sq = pl.BlockSpec((D, D), lambda ph, i: (0, 0))
    return pl.pallas_call(
        body, grid=(2, nblk),
        in_specs=[blk] * 4 + [one] * 2 +
                 [pl.BlockSpec((NB, 1), lambda ph, i: (i, 0))] +
                 [sq, one, sq, one],
        out_specs=pl.BlockSpec((G, D), lambda ph, i: (0, 0)),
        out_shape=jax.ShapeDtypeStruct((G, D), jnp.float32),
        scratch_shapes=[pltpu.VMEM((G, D), jnp.float32)] * 2 +
                       [pltpu.VMEM((1, D), jnp.float32)] * 2,
    )(x, ax, num, den, bn_g, bn_b, bids, w1, b1, w2, b2)


def _layer_weights(params, l):
    aw, bw = params[f"A_w{l}"], params[f"B_w{l}"]
    dw, ew = params[f"Dm_w{l}"], params[f"Em_w{l}"]
    ab, bb = params[f"A_b{l}"], params[f"B_b{l}"]
    db, eb = params[f"Dm_b{l}"], params[f"Em_b{l}"]
    wc = jnp.concatenate(
        [aw, dw, ew[:, :HALF], bw[:, :HALF], ew[:, HALF:], bw[:, HALF:]],
        axis=1)
    bc = jnp.concatenate(
        [ab, db, eb[:HALF], bb[:HALF], eb[HALF:], bb[HALF:]]).reshape(1, 4 * D)
    return wc, bc


def kernel(x, edge_attr, edge_index, batch_ids, params):
    del edge_attr
    n, d = x.shape
    assert d == D
    e = edge_index.shape[1]
    NP = ((n + 1 + NB - 1) // NB) * NB
    EPQ = 2 * TILES * W
    EP = ((e + EPQ - 1) // EPQ) * EPQ

    src = edge_index[0].astype(jnp.int32)
    dst = edge_index[1].astype(jnp.int32)
    pad_e = jnp.full((EP - e,), n, jnp.int32)
    srcp = jnp.concatenate([src, pad_e])
    dstp = jnp.concatenate([dst, pad_e])

    h = jnp.zeros((NP, D), jnp.float32).at[:n].set(x)
    bids = jnp.concatenate(
        [batch_ids.astype(jnp.int32), jnp.full((NP - n,), G, jnp.int32)]
    ).reshape(NP, 1)

    edge = _make_edge_kernel(NP, EP)

    def run_edge(ebt, dx):
        num, den = edge(srcp, dstp,
                        ebt.reshape(2 * NP, 2 * HALF),
                        dx.reshape(2 * NP, HALF))
        return num.reshape(NP, D), den.reshape(NP, D)

    wc0, bc0 = _layer_weights(params, 0)
    wc1, bc1 = _layer_weights(params, 1)

    ax0, dx0, ebt0 = _matmul4(h, wc0, bc0)
    num0, den0 = run_edge(ebt0, dx0)
    h1, ax1, dx1, ebt1 = _fused_mid(
        h, ax0, num0, den0,
        params["bn_g0"].reshape(1, D), params["bn_b0"].reshape(1, D),
        wc1, bc1, n)
    num1, den1 = run_edge(ebt1, dx1)
    return _fused_final(
        h1, ax1, num1, den1,
        params["bn_g1"].reshape(1, D), params["bn_b1"].reshape(1, D),
        bids,
        params["post_w1"], params["post_b1"].reshape(1, D),
        params["post_w2"], params["post_b2"].reshape(1, D), n)


# speedup vs baseline: 4.0479x; 4.0479x over previous
import functools

import jax
import jax.numpy as jnp
from jax import lax
from jax.experimental import pallas as pl
from jax.experimental.pallas import tpu as pltpu
from jax.experimental.pallas import tpu_sc as plsc

D = 128
G = 64
HALF = 64
W = 64
TILES = 16
NB = 512


@functools.lru_cache(maxsize=None)
def _make_edge_kernel(NP, EP):
    per_tile = EP // TILES
    chunks = per_tile // W
    assert chunks % 2 == 0 and chunks >= 4
    rows_pt = NP // TILES
    mesh = plsc.VectorSubcoreMesh(core_axis_name="c", subcore_axis_name="s")
    out_t = (jax.ShapeDtypeStruct((NP, 2, HALF), jnp.float32),
             jax.ShapeDtypeStruct((NP, 2, HALF), jnp.float32))
    idx_t = pltpu.VMEM((W,), jnp.int32)
    ctr_t = pltpu.VMEM((W, HALF), jnp.float32)
    scratch = (
        [idx_t] * 4 +
        [idx_t] * 8 +
        [ctr_t] * 6 +
        [ctr_t] * 4 +
        [pltpu.VMEM((64, HALF), jnp.float32)] +
        [pltpu.VMEM_SHARED((NP, HALF), jnp.float32)] * 2 +
        [pltpu.SemaphoreType.DMA] * 6
    )

    @functools.partial(
        pl.kernel, out_type=out_t, mesh=mesh, scratch_types=scratch,
        compiler_params=pltpu.CompilerParams(use_tc_tiling_on_sc=False))
    def edge_kernel(srcp, dstp, ex2, dx2, bx2, num_out, den_out,
                    isr0, isr1, idr0, idr1,
                    i2s0, i2s1, i2d0, i2d1, sd0, sd1, ssd0, ssd1,
                    rE0, rE1, rD0, rD1, rB0, rB1,
                    cn0, cn1, cd0, cd1, zb,
                    accn, accd, semG0, semG1, semI0, semI1, semS0, semS1):
        israw = (isr0, isr1)
        idraw = (idr0, idr1)
        i2s = (i2s0, i2s1)
        i2d = (i2d0, i2d1)
        sd = (sd0, sd1)
        ssd = (ssd0, ssd1)
        rE = (rE0, rE1)
        rD = (rD0, rD1)
        rB = (rB0, rB1)
        cn = (cn0, cn1)
        cd = (cd0, cd1)
        semG = (semG0, semG1)
        semI = (semI0, semI1)
        semS = (semS0, semS1)

        c = lax.axis_index("c")
        s = lax.axis_index("s")

        zv = jnp.zeros((16,), jnp.float32)

        def zrow(i, _):
            for f in range(HALF // 16):
                zb[i, pl.ds(f * 16, 16)] = zv
            return 0

        lax.fori_loop(0, 64, zrow, 0)
        r0 = s * rows_pt
        for k in range(rows_pt // 64):
            pltpu.sync_copy(zb, accn.at[pl.ds(r0 + k * 64, 64)])
            pltpu.sync_copy(zb, accd.at[pl.ds(r0 + k * 64, 64)])
        plsc.subcore_barrier()

        base0 = s * per_tile

        def prep(b):
            for q in range(W // 16):
                sl = pl.ds(q * 16, 16)
                dv = idraw[b][sl]
                i2s[b][sl] = israw[b][sl] * 2 + c
                i2d[b][sl] = dv * 2 + c
                sd[b][sl] = dv

        def issue_gathers(b):
            pltpu.async_copy(ex2.at[i2s[b]], rE[b], semG[b])
            pltpu.async_copy(dx2.at[i2d[b]], rD[b], semG[b])
            pltpu.async_copy(bx2.at[i2s[b]], rB[b], semG[b])

        def issue_idx(b, g):
            off = pl.multiple_of(base0 + g * W, W)
            pltpu.async_copy(srcp.at[pl.ds(off, W)], israw[b], semI[b])
            pltpu.async_copy(dstp.at[pl.ds(off, W)], idraw[b], semI[b])

        def drain_gathers(b):
            pltpu.make_async_copy(ex2.at[pl.ds(0, W)], rE[b], semG[b]).wait()
            pltpu.make_async_copy(dx2.at[pl.ds(0, W)], rD[b], semG[b]).wait()
            pltpu.make_async_copy(bx2.at[pl.ds(0, W)], rB[b], semG[b]).wait()

        def drain_idx(b):
            pltpu.make_async_copy(srcp.at[pl.ds(0, W)], israw[b], semI[b]).wait()
            pltpu.make_async_copy(dstp.at[pl.ds(0, W)], idraw[b], semI[b]).wait()

        def drain_scatters(b):
            pltpu.make_async_copy(num_out.at[pl.ds(0, W), 0], cn[b],
                                  semS[b]).wait()
            pltpu.make_async_copy(num_out.at[pl.ds(0, W), 0], cd[b],
                                  semS[b]).wait()

        def compute(b):
            def ew(w, _):
                for f in range(HALF // 16):
                    sl = pl.ds(f * 16, 16)
                    t = rD[b][w, sl] + rE[b][w, sl]
                    sg = 1.0 / (1.0 + jnp.exp(-t))
                    cd[b][w, sl] = sg
                    cn[b][w, sl] = sg * rB[b][w, sl]
                return 0

            lax.fori_loop(0, W, ew, 0)

        for b in (0, 1):
            off = pl.multiple_of(base0 + b * W, W)
            pltpu.sync_copy(srcp.at[pl.ds(off, W)], israw[b])
            pltpu.sync_copy(dstp.at[pl.ds(off, W)], idraw[b])
            prep(b)
            issue_gathers(b)
        issue_idx(0, 2)
        issue_idx(1, 3)

        def pair(p, _):
            for b in (0, 1):
                g = p * 2 + b
                drain_gathers(b)

                @pl.when(g >= 2)
                def _():
                    drain_scatters(b)

                compute(b)
                for q in range(W // 16):
                    sl = pl.ds(q * 16, 16)
                    ssd[b][sl] = sd[b][sl]
                pltpu.async_copy(cn[b], accn.at[ssd[b]], semS[b], add=True)
                pltpu.async_copy(cd[b], accd.at[ssd[b]], semS[b], add=True)

                @pl.when(g + 2 < chunks)
                def _():
                    drain_idx(b)
                    prep(b)
                    issue_gathers(b)

                @pl.when(g + 4 < chunks)
                def _():
                    issue_idx(b, g + 4)
            return 0

        lax.fori_loop(0, chunks // 2, pair, 0)
        drain_scatters(0)
        drain_scatters(1)
        plsc.subcore_barrier()

        for k in range(rows_pt // 64):
            rr = pl.ds(r0 + k * 64, 64)
            pltpu.sync_copy(accn.at[rr], num_out.at[rr, c])
            pltpu.sync_copy(accd.at[rr], den_out.at[rr, c])

    return edge_kernel


def _split_tables(y):
    return (y[:, 0 * D:1 * D], y[:, 1 * D:2 * D],
            y[:, 2 * D:3 * D], y[:, 3 * D:4 * D])


def _matmul4(h, wc, bc):
    NP = h.shape[0]

    def body(h_ref, w_ref, b_ref, oa, ob, od, oe):
        y = jnp.dot(h_ref[...], w_ref[...],
                    preferred_element_type=jnp.float32) + b_ref[...]
        ax, bx, dx, ex = _split_tables(y)
        oa[...] = ax
        ob[...] = bx
        od[...] = dx
        oe[...] = ex

    out = [jax.ShapeDtypeStruct((NP, D), jnp.float32)] * 4
    return pl.pallas_call(
        body, grid=(NP // NB,),
        in_specs=[pl.BlockSpec((NB, D), lambda i: (i, 0)),
                  pl.BlockSpec((D, 4 * D), lambda i: (0, 0)),
                  pl.BlockSpec((1, 4 * D), lambda i: (0, 0))],
        out_specs=[pl.BlockSpec((NB, D), lambda i: (i, 0))] * 4,
        out_shape=out)(h, wc, bc)


def _hn_and_mask(a_ref, n_ref, d_ref, i, n_valid):
    hn = a_ref[...] + n_ref[...] / (d_ref[...] + 1e-6)
    row = lax.broadcasted_iota(jnp.int32, (NB, 1), 0) + i * NB
    return hn, row < n_valid


def _bn_apply(x_ref, hn, mask, sums, sumsq, g_ref, b_ref, inv_n):
    mean = sums[...] * inv_n
    var = sumsq[...] * inv_n - mean * mean
    y = (hn - mean) * lax.rsqrt(var + 1e-5) * g_ref[...] + b_ref[...]
    h = x_ref[...] + jnp.maximum(y, 0.0)
    return jnp.where(mask, h, 0.0)


def _fused_mid(x, ax, num, den, bn_g, bn_b, wc, bc, n_valid):
    NP = x.shape[0]
    inv_n = 1.0 / n_valid

    def body(x_ref, a_ref, n_ref, d_ref, g_ref, b_ref, w_ref, bc_ref,
             oh, oa, ob, od, oe, sums, sumsq):
        ph = pl.program_id(0)
        i = pl.program_id(1)
        hn, mask = _hn_and_mask(a_ref, n_ref, d_ref, i, n_valid)

        @pl.when(ph == 0)
        def _():
            @pl.when(i == 0)
            def _():
                sums[...] = jnp.zeros_like(sums)
                sumsq[...] = jnp.zeros_like(sumsq)

            hm = jnp.where(mask, hn, 0.0)
            sums[...] += jnp.sum(hm, axis=0, keepdims=True)
            sumsq[...] += jnp.sum(hm * hm, axis=0, keepdims=True)

        @pl.when(ph == 1)
        def _():
            h = _bn_apply(x_ref, hn, mask, sums, sumsq, g_ref, b_ref, inv_n)
            oh[...] = h
            y = jnp.dot(h, w_ref[...],
                        preferred_element_type=jnp.float32) + bc_ref[...]
            ax2, bx2, dx2, ex2 = _split_tables(y)
            oa[...] = ax2
            ob[...] = bx2
            od[...] = dx2
            oe[...] = ex2

    blk = pl.BlockSpec((NB, D), lambda ph, i: (i, 0))
    one = pl.BlockSpec((1, D), lambda ph, i: (0, 0))
    out = [jax.ShapeDtypeStruct((NP, D), jnp.float32)] * 5
    return pl.pallas_call(
        body, grid=(2, NP // NB),
        in_specs=[blk] * 4 + [one] * 2 +
                 [pl.BlockSpec((D, 4 * D), lambda ph, i: (0, 0)),
                  pl.BlockSpec((1, 4 * D), lambda ph, i: (0, 0))],
        out_specs=[blk] * 5,
        out_shape=out,
        scratch_shapes=[pltpu.VMEM((1, D), jnp.float32)] * 2,
    )(x, ax, num, den, bn_g, bn_b, wc, bc)


def _fused_final(x, ax, num, den, bn_g, bn_b, bids, w1, b1, w2, b2, n_valid):
    NP = x.shape[0]
    inv_n = 1.0 / n_valid
    nblk = NP // NB

    def body(x_ref, a_ref, n_ref, d_ref, g_ref, b_ref, bid_ref,
             w1_ref, b1_ref, w2_ref, b2_ref, out_ref, psum, pcnt,
             sums, sumsq):
        ph = pl.program_id(0)
        i = pl.program_id(1)
        hn, mask = _hn_and_mask(a_ref, n_ref, d_ref, i, n_valid)

        @pl.when(ph == 0)
        def _():
            @pl.when(i == 0)
            def _():
                sums[...] = jnp.zeros_like(sums)
                sumsq[...] = jnp.zeros_like(sumsq)

            hm = jnp.where(mask, hn, 0.0)
            sums[...] += jnp.sum(hm, axis=0, keepdims=True)
            sumsq[...] += jnp.sum(hm * hm, axis=0, keepdims=True)

        @pl.when(ph == 1)
        def _():
            h = _bn_apply(x_ref, hn, mask, sums, sumsq, g_ref, b_ref, inv_n)
            oh = (bid_ref[...] ==
                  lax.broadcasted_iota(jnp.int32, (1, G), 1))
            oh = oh.astype(jnp.float32)

            @pl.when(i == 0)
            def _():
                psum[...] = jnp.zeros_like(psum)
                pcnt[...] = jnp.zeros_like(pcnt)

            dn = (((0,), (0,)), ((), ()))
            psum[...] += lax.dot_general(oh, h, dn,
                                         preferred_element_type=jnp.float32)
            pcnt[...] += lax.dot_general(oh, jnp.ones((NB, D), jnp.float32),
                                         dn, preferred_element_type=jnp.float32)

            @pl.when(i == nblk - 1)
            def _():
                gm = psum[...] / jnp.maximum(pcnt[...], 1.0)
                t = jnp.dot(gm, w1_ref[...],
                            preferred_element_type=jnp.float32)
                t = jnp.maximum(t + b1_ref[...], 0.0)
                out_ref[...] = jnp.dot(
                    t, w2_ref[...],
                    preferred_element_type=jnp.float32) + b2_ref[...]

    blk = pl.BlockSpec((NB, D), lambda ph, i: (i, 0))
    one = pl.BlockSpec((1, D), lambda ph, i: (0, 0))
    sq = pl.BlockSpec((D, D), lambda ph, i: (0, 0))
    return pl.pallas_call(
        body, grid=(2, nblk),
        in_specs=[blk] * 4 + [one] * 2 +
                 [pl.BlockSpec((NB, 1), lambda ph, i: (i, 0))] +
                 [sq, one, sq, one],
        out_specs=pl.BlockSpec((G, D), lambda ph, i: (0, 0)),
        out_shape=jax.ShapeDtypeStruct((G, D), jnp.float32),
        scratch_shapes=[pltpu.VMEM((G, D), jnp.float32)] * 2 +
                       [pltpu.VMEM((1, D), jnp.float32)] * 2,
    )(x, ax, num, den, bn_g, bn_b, bids, w1, b1, w2, b2)


def _layer_weights(params, l):
    wc = jnp.concatenate([params[f"{nm}_w{l}"]
                          for nm in ("A", "B", "Dm", "Em")], axis=1)
    bc = jnp.concatenate([params[f"{nm}_b{l}"]
                          for nm in ("A", "B", "Dm", "Em")]).reshape(1, 4 * D)
    return wc, bc


def kernel(x, edge_attr, edge_index, batch_ids, params):
    del edge_attr
    n, d = x.shape
    assert d == D
    e = edge_index.shape[1]
    NP = ((n + 1 + NB - 1) // NB) * NB
    EPQ = 2 * TILES * W
    EP = ((e + EPQ - 1) // EPQ) * EPQ

    src = edge_index[0].astype(jnp.int32)
    dst = edge_index[1].astype(jnp.int32)
    pad_e = jnp.full((EP - e,), n, jnp.int32)
    srcp = jnp.concatenate([src, pad_e])
    dstp = jnp.concatenate([dst, pad_e])

    h = jnp.zeros((NP, D), jnp.float32).at[:n].set(x)
    bids = jnp.concatenate(
        [batch_ids.astype(jnp.int32), jnp.full((NP - n,), G, jnp.int32)]
    ).reshape(NP, 1)

    edge = _make_edge_kernel(NP, EP)

    def run_edge(ex, dx, bx):
        num, den = edge(srcp, dstp,
                        ex.reshape(2 * NP, HALF),
                        dx.reshape(2 * NP, HALF),
                        bx.reshape(2 * NP, HALF))
        return num.reshape(NP, D), den.reshape(NP, D)

    wc0, bc0 = _layer_weights(params, 0)
    wc1, bc1 = _layer_weights(params, 1)

    ax0, bx0, dx0, ex0 = _matmul4(h, wc0, bc0)
    num0, den0 = run_edge(ex0, dx0, bx0)
    h1, ax1, bx1, dx1, ex1 = _fused_mid(
        h, ax0, num0, den0,
        params["bn_g0"].reshape(1, D), params["bn_b0"].reshape(1, D),
        wc1, bc1, n)
    num1, den1 = run_edge(ex1, dx1, bx1)
    return _fused_final(
        h1, ax1, num1, den1,
        params["bn_g1"].reshape(1, D), params["bn_b1"].reshape(1, D),
        bids,
        params["post_w1"], params["post_b1"].reshape(1, D),
        params["post_w2"], params["post_b2"].reshape(1, D), n)
